# Initial kernel scaffold; baseline (speedup 1.0000x reference)
import jax
import jax.numpy as jnp
from jax.experimental import pallas as pl


def kernel(x, y):
    raise NotImplementedError("write your pallas kernel here")


# speedup vs baseline: 157.8679x; 157.8679x over previous
import jax
import jax.numpy as jnp
from jax import lax
from jax.experimental import pallas as pl
from jax.experimental.pallas import tpu as pltpu
from jax.experimental.pallas import tpu_sc as plsc

_NC, _NS, _L = 2, 16, 16
_NW = _NC * _NS

_ROWS, _COLS = 16384, 200
_N = _ROWS * _COLS
_PER_W = _N // _NW
_VECS = _PER_W // _L
_MAGIC = 12582912.0


def _sc_body(x_hbm, y_hbm, o_hbm, y_v, buf):
    wid = lax.axis_index("s") * _NC + lax.axis_index("c")
    base = wid * _PER_W
    pltpu.sync_copy(y_hbm, y_v)
    pltpu.sync_copy(x_hbm.at[pl.ds(base, _PER_W)], buf)

    scale = jnp.float32(y_v.shape[0] - 1)

    def body(i, carry):
        sl = pl.ds(i * _L, _L)
        t = (buf[sl] * scale + _MAGIC) - _MAGIC
        idx = t.astype(jnp.int32)
        buf[sl] = plsc.load_gather(y_v, [idx])
        return carry

    lax.fori_loop(0, _VECS, body, 0)
    pltpu.sync_copy(buf, o_hbm.at[pl.ds(base, _PER_W)])


_sc_call = pl.kernel(
    _sc_body,
    out_type=jax.ShapeDtypeStruct((_N,), jnp.float32),
    mesh=plsc.VectorSubcoreMesh(core_axis_name="c", subcore_axis_name="s"),
    scratch_types=[
        pltpu.VMEM((128,), jnp.float32),
        pltpu.VMEM((_PER_W,), jnp.float32),
    ],
    compiler_params=pltpu.CompilerParams(needs_layout_passes=False),
)


def kernel(x, y):
    out = _sc_call(x.reshape(_N).astype(y.dtype), y)
    return out.reshape(x.shape)


# speedup vs baseline: 284.2405x; 1.8005x over previous
import jax
import jax.numpy as jnp
from jax import lax
from jax.experimental import pallas as pl
from jax.experimental.pallas import tpu as pltpu
from jax.experimental.pallas import tpu_sc as plsc

_NC, _NS, _L = 2, 16, 16
_NW = _NC * _NS

_ROWS, _COLS = 16384, 200
_N = _ROWS * _COLS
_PER_W = _N // _NW
_VECS = _PER_W // _L
_MAGIC = 12582912.0


def _sc_body(x_hbm, y_hbm, o_hbm, y_v, buf):
    wid = lax.axis_index("s") * _NC + lax.axis_index("c")
    base = wid * _PER_W
    pltpu.sync_copy(y_hbm, y_v)
    pltpu.sync_copy(x_hbm.at[pl.ds(base, _PER_W)], buf)

    scale = jnp.float32(y_v.shape[0] - 1)

    @plsc.parallel_loop(0, _PER_W, step=_L, unroll=8)
    def body(i):
        sl = pl.ds(i, _L)
        t = (buf[sl] * scale + _MAGIC) - _MAGIC
        idx = t.astype(jnp.int32)
        buf[sl] = plsc.load_gather(y_v, [idx])
    pltpu.sync_copy(buf, o_hbm.at[pl.ds(base, _PER_W)])


_sc_call = pl.kernel(
    _sc_body,
    out_type=jax.ShapeDtypeStruct((_N,), jnp.float32),
    mesh=plsc.VectorSubcoreMesh(core_axis_name="c", subcore_axis_name="s"),
    scratch_types=[
        pltpu.VMEM((128,), jnp.float32),
        pltpu.VMEM((_PER_W,), jnp.float32),
    ],
    compiler_params=pltpu.CompilerParams(needs_layout_passes=False),
)


def kernel(x, y):
    out = _sc_call(x.reshape(_N).astype(y.dtype), y)
    return out.reshape(x.shape)
